# Initial kernel scaffold; baseline (speedup 1.0000x reference)
import jax
import jax.numpy as jnp
from jax.experimental import pallas as pl


def kernel(positions, inv_freq):
    raise NotImplementedError("write your pallas kernel here")


# speedup vs baseline: 1.2413x; 1.2413x over previous
import functools

import jax
import jax.numpy as jnp
from jax import lax
from jax.experimental import pallas as pl
from jax.experimental.pallas import tpu as pltpu
from jax.experimental.pallas import tpu_sc as plsc

DIM_HALF = 64
DC = 2 * DIM_HALF
EXT = 9216
SEQ = 8192
ROW_BLK = 1024
NUM_BLKS = EXT // ROW_BLK

NC = 2
NS = 16
NW = NC * NS
BPW = SEQ // NW


def _cache_body(invf_ref, out_ref):
    i = pl.program_id(0)
    row0 = (i * ROW_BLK).astype(jnp.float32)
    rows = (lax.broadcasted_iota(jnp.int32, (ROW_BLK, DIM_HALF), 0)
            .astype(jnp.float32) + row0)
    ang = rows * invf_ref[...]
    out_ref[...] = jnp.concatenate([jnp.cos(ang), jnp.sin(ang)], axis=1)


def _build_cache(inv_freq):
    invf2d = inv_freq.reshape(1, DIM_HALF)
    return pl.pallas_call(
        _cache_body,
        grid=(NUM_BLKS,),
        in_specs=[pl.BlockSpec((1, DIM_HALF), lambda i: (0, 0))],
        out_specs=pl.BlockSpec((ROW_BLK, DC), lambda i: (i, 0)),
        out_shape=jax.ShapeDtypeStruct((EXT, DC), jnp.float32),
    )(invf2d)


@functools.cache
def _make_sc_gather():
    mesh = plsc.VectorSubcoreMesh(core_axis_name="c", subcore_axis_name="s")

    @functools.partial(
        pl.kernel,
        mesh=mesh,
        out_type=jax.ShapeDtypeStruct((SEQ, DC), jnp.float32),
        scratch_types=[
            pltpu.VMEM((BPW,), jnp.int32),
            pltpu.VMEM((BPW, DC), jnp.float32),
            pltpu.SemaphoreType.DMA,
        ],
    )
    def _sc_gather(cache_hbm, pos_hbm, out_hbm, idx_v, rows_v, sem):
        wid = lax.axis_index("s") * NC + lax.axis_index("c")
        base = wid * BPW
        pltpu.sync_copy(pos_hbm.at[pl.ds(base, BPW)], idx_v)
        pltpu.async_copy(cache_hbm.at[idx_v], rows_v, sem).wait()
        pltpu.sync_copy(rows_v, out_hbm.at[pl.ds(base, BPW)])

    return _sc_gather


def kernel(positions, inv_freq):
    cache = _build_cache(inv_freq)
    pos32 = positions.astype(jnp.int32)
    both = _make_sc_gather()(cache, pos32)
    return (both[:, :DIM_HALF], both[:, DIM_HALF:])
